# baseline (speedup 1.0000x reference)
import functools

import jax
import jax.numpy as jnp
from jax import lax
from jax.experimental import pallas as pl
from jax.experimental.pallas import tpu as pltpu
from jax.experimental.pallas import tpu_sc as plsc

BATCH = 16384
EMBED_DIM = 32
NUM_CORES = 2
NUM_SUBCORES = 16
NUM_WORKERS = NUM_CORES * NUM_SUBCORES
BPW = BATCH // NUM_WORKERS


def _make_kg_gather():
    mesh = plsc.VectorSubcoreMesh(core_axis_name="c", subcore_axis_name="s")

    @functools.partial(
        pl.kernel,
        mesh=mesh,
        compiler_params=pltpu.CompilerParams(use_tc_tiling_on_sc=False),
        out_type=jax.ShapeDtypeStruct((3, BATCH, EMBED_DIM), jnp.float32),
        scratch_types=[
            pltpu.VMEM((BPW,), jnp.int32),
            pltpu.VMEM((BPW,), jnp.int32),
            pltpu.VMEM((BPW,), jnp.int32),
            pltpu.VMEM((BPW, EMBED_DIM), jnp.float32),
            pltpu.VMEM((BPW, EMBED_DIM), jnp.float32),
            pltpu.VMEM((BPW, EMBED_DIM), jnp.float32),
            pltpu.SemaphoreType.DMA,
            pltpu.SemaphoreType.DMA,
            pltpu.SemaphoreType.DMA,
        ],
    )
    def kg_gather(h_hbm, r_hbm, t_hbm, ent_hbm, rel_hbm, out_hbm,
                  h_idx, r_idx, t_idx, h_rows, r_rows, t_rows,
                  sem_h, sem_r, sem_t):
        wid = lax.axis_index("s") * NUM_CORES + lax.axis_index("c")
        base = wid * BPW
        pltpu.sync_copy(h_hbm.at[pl.ds(base, BPW)], h_idx)
        pltpu.sync_copy(r_hbm.at[pl.ds(base, BPW)], r_idx)
        pltpu.sync_copy(t_hbm.at[pl.ds(base, BPW)], t_idx)
        ch = pltpu.async_copy(ent_hbm.at[h_idx], h_rows, sem_h)
        cr = pltpu.async_copy(rel_hbm.at[r_idx], r_rows, sem_r)
        ct = pltpu.async_copy(ent_hbm.at[t_idx], t_rows, sem_t)
        ch.wait()
        pltpu.sync_copy(h_rows, out_hbm.at[0, pl.ds(base, BPW)])
        cr.wait()
        pltpu.sync_copy(r_rows, out_hbm.at[1, pl.ds(base, BPW)])
        ct.wait()
        pltpu.sync_copy(t_rows, out_hbm.at[2, pl.ds(base, BPW)])

    return kg_gather


_kg_gather = _make_kg_gather()


@jax.jit
def kernel(h, r, t, entity_emb, relation_emb):
    h = h.astype(jnp.int32)
    r = r.astype(jnp.int32)
    t = t.astype(jnp.int32)
    return _kg_gather(h, r, t, entity_emb, relation_emb)


# speedup vs baseline: 2.7233x; 2.7233x over previous
import functools

import jax
import jax.numpy as jnp
from jax import lax
from jax.experimental import pallas as pl
from jax.experimental.pallas import tpu as pltpu
from jax.experimental.pallas import tpu_sc as plsc

NE = 1000000
NR = 1000
BATCH = 16384
EMBED_DIM = 32
NUM_CORES = 2
NUM_SUBCORES = 16
NW = NUM_CORES * NUM_SUBCORES
BPW = BATCH // NW
WIN = 512
TAIL0 = (NE // WIN) * WIN
NWIN_G = TAIL0 // WIN
KMAX = -(-NWIN_G // NW)
CHUNK = 2048
NCH = BATCH // CHUNK
PAIR_CAP = 2 * BATCH + 16 * KMAX + 16


def _iota16():
    return lax.iota(jnp.int32, 16)


def _bc(val):
    return jnp.full((16,), val, jnp.int32)


def _make_kg():
    mesh = plsc.VectorSubcoreMesh(core_axis_name="c", subcore_axis_name="s")

    @functools.partial(
        pl.kernel,
        mesh=mesh,
        compiler_params=pltpu.CompilerParams(
            use_tc_tiling_on_sc=True, needs_layout_passes=False),
        out_type=jax.ShapeDtypeStruct((3 * BATCH, 128), jnp.float32),
        scratch_types=[
            pltpu.VMEM((PAIR_CAP,), jnp.int32),
            pltpu.VMEM((2, 4, 8, WIN), jnp.float32),
            pltpu.VMEM((2, 128, 128), jnp.float32),
            pltpu.VMEM((2, 128), jnp.int32),
            pltpu.VMEM((CHUNK,), jnp.int32),
            pltpu.VMEM((16 * KMAX,), jnp.int32),
            pltpu.VMEM((16 * KMAX,), jnp.int32),
            pltpu.SemaphoreType.DMA,
            pltpu.SemaphoreType.DMA,
            pltpu.SemaphoreType.DMA,
            pltpu.SemaphoreType.DMA,
            pltpu.SemaphoreType.DMA,
        ],
    )
    def kg(h_hbm, r_hbm, t_hbm, ent3_hbm, relpad_hbm, tailpad_hbm, out_hbm,
           sorted_v, slab, staging, orow, ibuf, cnts, offs,
           sem_s0, sem_s1, sem_o0, sem_o1, sem_m):
        wid = lax.axis_index("s") * NUM_CORES + lax.axis_index("c")
        i16 = _iota16()

        bbase = wid * BPW
        for c in range(BPW // 128):
            pltpu.sync_copy(r_hbm.at[pl.ds(bbase + c * 128, 128)],
                            orow.at[0])
            pltpu.async_copy(relpad_hbm.at[orow.at[0]], staging.at[0],
                             sem_m).wait()
            pltpu.sync_copy(
                staging.at[0],
                out_hbm.at[pl.ds(BATCH + bbase + c * 128, 128), :])

        for j, arr in ((0, h_hbm), (1, t_hbm)):
            for c in range(BPW // 128):
                pltpu.sync_copy(arr.at[pl.ds(bbase + c * 128, 128)],
                                ibuf.at[pl.ds(0, 128)])
                for v in range(8):
                    idxv = ibuf[pl.ds(v * 16, 16)]
                    tmask = idxv >= TAIL0
                    orow.at[0][pl.ds(v * 16, 16)] = jnp.where(
                        tmask, idxv - TAIL0, -1)
                pltpu.async_copy(
                    tailpad_hbm.at[plsc.Indices(orow.at[0],
                                                ignored_value=-1)],
                    staging.at[0], sem_m).wait()
                for v in range(8):
                    idxv = ibuf[pl.ds(v * 16, 16)]
                    tmask = idxv >= TAIL0
                    rowb = bbase + c * 128 + v * 16 + i16 + j * 2 * BATCH
                    orow.at[1][pl.ds(v * 16, 16)] = jnp.where(tmask, rowb, -1)
                pltpu.async_copy(
                    staging.at[0],
                    out_hbm.at[plsc.Indices(orow.at[1], ignored_value=-1)],
                    sem_m).wait()

        for v in range(KMAX):
            cnts[pl.ds(v * 16, 16)] = jnp.zeros((16,), jnp.int32)
        ones = jnp.ones((16,), jnp.int32)
        for j, arr in ((0, h_hbm), (1, t_hbm)):
            for c in range(NCH):
                pltpu.sync_copy(arr.at[pl.ds(c * CHUNK, CHUNK)], ibuf)

                def f1_body(v, carry):
                    idxv = ibuf[pl.ds(v * 16, 16)]
                    gw = lax.shift_right_logical(idxv, 9)
                    mine = ((gw & 31) == wid) & (idxv < TAIL0)
                    lw = lax.shift_right_logical(gw, 5)
                    plsc.addupdate_scatter(cnts, [lw * 16 + i16], ones,
                                           mask=mine)
                    return carry

                lax.fori_loop(0, CHUNK // 16, f1_body, 0)

        def off_body(w, base16):
            lc = cnts[pl.ds(w * 16, 16)]
            cum = plsc.cumsum(lc)
            tot = jnp.zeros((16,), jnp.int32) + jnp.sum(lc)
            offs[pl.ds(w * 16, 16)] = base16 + cum - lc
            return base16 + ((tot + 15) & ~15)

        lax.fori_loop(0, KMAX, off_body, jnp.zeros((16,), jnp.int32))

        for j, arr in ((0, h_hbm), (1, t_hbm)):
            for c in range(NCH):
                pltpu.sync_copy(arr.at[pl.ds(c * CHUNK, CHUNK)], ibuf)

                def f2_body(v, carry, _j=j, _c=c):
                    idxv = ibuf[pl.ds(v * 16, 16)]
                    gw = lax.shift_right_logical(idxv, 9)
                    mine = ((gw & 31) == wid) & (idxv < TAIL0)
                    lw = lax.shift_right_logical(gw, 5)
                    addr = lw * 16 + i16
                    b = _c * CHUNK + v * 16 + i16
                    packed = (lax.shift_left(lw, _bc(24))
                              | lax.shift_left(idxv & 511, _bc(15))
                              | (_j << 14) | b)
                    dest = plsc.load_gather(offs, [addr])
                    plsc.store_scatter(sorted_v, [dest], packed, mask=mine)
                    plsc.addupdate_scatter(offs, [addr], ones, mask=mine)
                    return carry

                lax.fori_loop(0, CHUNK // 16, f2_body, 0)

        def issue_slab(k):
            g = wid + NW * k
            ok = g < NWIN_G

            @pl.when((k % 2 == 0) & ok)
            def _():
                pltpu.async_copy(
                    ent3_hbm.at[:, :, pl.ds(g * WIN, WIN)], slab.at[0],
                    sem_s0)

            @pl.when((k % 2 == 1) & ok)
            def _():
                pltpu.async_copy(
                    ent3_hbm.at[:, :, pl.ds(g * WIN, WIN)], slab.at[1],
                    sem_s1)

        issue_slab(0)
        issue_slab(1)

        def win_body(k, state):
            mstart, scat_flags = state
            p = k % 2
            g = wid + NW * k
            wvalid = g < NWIN_G

            @pl.when(wvalid & (p == 0))
            def _():
                pltpu.make_async_copy(
                    ent3_hbm.at[:, :, pl.ds(0, WIN)], slab.at[0],
                    sem_s0).wait()

            @pl.when(wvalid & (p == 1))
            def _():
                pltpu.make_async_copy(
                    ent3_hbm.at[:, :, pl.ds(0, WIN)], slab.at[1],
                    sem_s1).wait()

            cnt = jnp.sum(cnts[pl.ds(k * 16, 16)])
            nchk = lax.div(cnt + 127, 128)

            def chunk_body(q, flags):
                f0, f1 = flags

                @pl.when(((q > 0) | (f0 > 0)) & (p == 0))
                def _():
                    pltpu.make_async_copy(
                        staging.at[0],
                        out_hbm.at[plsc.Indices(orow.at[0],
                                                ignored_value=-1)],
                        sem_o0).wait()

                @pl.when(((q > 0) | (f1 > 0)) & (p == 1))
                def _():
                    pltpu.make_async_copy(
                        staging.at[1],
                        out_hbm.at[plsc.Indices(orow.at[1],
                                                ignored_value=-1)],
                        sem_o1).wait()

                rem = cnt - q * 128

                def vec_body(v, carry):
                    pkv = sorted_v[pl.ds(mstart + q * 128 + v * 16, 16)]
                    valid = (v * 16 + i16) < rem
                    lvec = lax.shift_right_logical(pkv, 15) & 511
                    rowid = (pkv & 16383) + \
                        (lax.shift_right_logical(pkv, 14) & 1) * (2 * BATCH)
                    orow.at[p][pl.ds(v * 16, 16)] = jnp.where(
                        valid, rowid, -1)
                    rows = v * 16 + i16
                    for d in range(EMBED_DIM):
                        vals = plsc.load_gather(
                            slab.at[p], [_bc(d // 8), _bc(d % 8), lvec])
                        plsc.store_scatter(staging.at[p], [rows, _bc(d)],
                                           vals)
                    return carry

                nvec = lax.div(jnp.maximum(rem, 0) + 15, 16)
                lax.fori_loop(0, jnp.minimum(nvec, 8), vec_body, 0)

                def pad_body(v, carry):
                    orow.at[p][pl.ds(v * 16, 16)] = _bc(-1)
                    return carry

                lax.fori_loop(jnp.minimum(nvec, 8), 8, pad_body, 0)

                @pl.when(p == 0)
                def _():
                    pltpu.async_copy(
                        staging.at[0],
                        out_hbm.at[plsc.Indices(orow.at[0],
                                                ignored_value=-1)],
                        sem_o0)

                @pl.when(p == 1)
                def _():
                    pltpu.async_copy(
                        staging.at[1],
                        out_hbm.at[plsc.Indices(orow.at[1],
                                                ignored_value=-1)],
                        sem_o1)

                nf0 = jnp.where(p == 0, 1, f0)
                nf1 = jnp.where(p == 1, 1, f1)
                return nf0, nf1

            new_flags = lax.cond(
                wvalid & (nchk > 0),
                lambda: lax.fori_loop(0, nchk, chunk_body, scat_flags),
                lambda: scat_flags)
            issue_slab(k + 2)
            return (mstart + ((cnt + 15) & ~15), new_flags)

        _, (f0, f1) = lax.fori_loop(
            0, KMAX, win_body, (jnp.int32(0), (jnp.int32(0), jnp.int32(0))))

        @pl.when(f0 > 0)
        def _():
            pltpu.make_async_copy(
                staging.at[0],
                out_hbm.at[plsc.Indices(orow.at[0], ignored_value=-1)],
                sem_o0).wait()

        @pl.when(f1 > 0)
        def _():
            pltpu.make_async_copy(
                staging.at[1],
                out_hbm.at[plsc.Indices(orow.at[1], ignored_value=-1)],
                sem_o1).wait()

    return kg


_kg = _make_kg()


@jax.jit
def kernel(h, r, t, entity_emb, relation_emb):
    h = h.astype(jnp.int32)
    r = r.astype(jnp.int32)
    t = t.astype(jnp.int32)
    ent3 = entity_emb.T.reshape(4, 8, NE)
    relpad = jnp.pad(relation_emb, ((0, 0), (0, 128 - EMBED_DIM)))
    tailpad = jnp.pad(entity_emb[TAIL0:, :],
                      ((0, 128 - (NE - TAIL0)), (0, 128 - EMBED_DIM)))
    out = _kg(h, r, t, ent3, relpad, tailpad)
    return out.reshape(3, BATCH, 128)[:, :, :EMBED_DIM]
